# baseline (speedup 1.0000x reference)
import functools

import jax
import jax.numpy as jnp
from jax import lax
from jax.experimental import pallas as pl
from jax.experimental.pallas import tpu as pltpu
from jax.experimental.pallas import tpu_sc as plsc

EMB = 32
BATCH = 16384
HIST = 200
NC, NS = 2, 16
NW = NC * NS
BT = BATCH // 128
BT_PER_W = BT // NW
CW = BT_PER_W * 128
TW = BT_PER_W * 128 * EMB
HROW = BT * 128 * EMB

_mesh = plsc.VectorSubcoreMesh(core_axis_name="c", subcore_axis_name="s")


@functools.partial(
    pl.kernel,
    mesh=_mesh,
    out_type=jax.ShapeDtypeStruct((HIST, HROW), jnp.float32),
    scratch_types=[
        pltpu.VMEM((2, CW), jnp.int32),
        pltpu.VMEM((2, CW, EMB), jnp.float32),
        pltpu.VMEM((2, TW), jnp.float32),
        pltpu.SemaphoreType.DMA,
        pltpu.SemaphoreType.DMA,
        pltpu.SemaphoreType.DMA,
        pltpu.SemaphoreType.DMA,
        pltpu.SemaphoreType.DMA,
        pltpu.SemaphoreType.DMA,
    ],
    compiler_params=pltpu.CompilerParams(use_tc_tiling_on_sc=False,
                                         needs_layout_passes=False),
)
def _emb_lookup(idx_hbm, table_hbm, out_hbm, idx_v, g_v, t_v,
                isem0, isem1, gsem0, gsem1, osem0, osem1):
    wid = lax.axis_index("s") * NC + lax.axis_index("c")
    col0 = wid * CW
    bt0 = wid * BT_PER_W
    isem = (isem0, isem1)
    gsem = (gsem0, gsem1)
    osem = (osem0, osem1)
    iota16 = jnp.arange(16, dtype=jnp.int32)
    pat_lo = (iota16 // 8) * 4096 + (iota16 % 8) * 128
    pat_hi = pat_lo + 2 * 4096

    def fire_gathers(h, b):
        pltpu.make_async_copy(idx_hbm.at[h, pl.ds(col0, CW)], idx_v.at[b],
                              isem[b]).wait()
        for j in range(BT_PER_W):
            pltpu.async_copy(
                table_hbm.at[idx_v.at[b, pl.ds(j * 128, 128)]],
                g_v.at[b, pl.ds(j * 128, 128)], gsem[b])

    def drain_gathers(b):
        for j in range(BT_PER_W):
            pltpu.make_async_copy(
                table_hbm.at[idx_v.at[b, pl.ds(j * 128, 128)]],
                g_v.at[b, pl.ds(j * 128, 128)], gsem[b]).wait()

    def transpose(c):
        @plsc.parallel_loop(0, CW // 16, unroll=2)
        def _(rb):
            rows = rb * 16 + iota16
            s = (rb >> 3) * 1024 + (rb & 7) * 16
            for dhalf in range(2):
                for d0 in range(16):
                    m = (d0 + 5 * iota16) & 15
                    cols = dhalf * 16 + m
                    vec = plsc.load_gather(g_v.at[c], [rows, cols])
                    wpat = (dhalf * 8192 + (m // 8) * 4096
                            + (m % 8) * 128 + iota16)
                    plsc.store_scatter(t_v.at[c], [wpat + s], vec)

    def store_out(h, c):
        for dt in range(EMB // 8):
            pltpu.async_copy(
                t_v.at[c, pl.ds(dt * 4096, 4096)],
                out_hbm.at[h, pl.ds(dt * (HROW // 4) + bt0 * 1024, 4096)],
                osem[c])

    def wait_store(h, c):
        for dt in range(EMB // 8):
            pltpu.make_async_copy(
                t_v.at[c, pl.ds(dt * 4096, 4096)],
                out_hbm.at[h, pl.ds(dt * (HROW // 4) + bt0 * 1024, 4096)],
                osem[c]).wait()

    pltpu.async_copy(idx_hbm.at[0, pl.ds(col0, CW)], idx_v.at[0], isem[0])

    def outer(t, carry):
        for b in range(2):
            h = t * 2 + b
            c = 1 - b
            fire_gathers(h, b)

            @pl.when(h >= 1)
            def _():
                drain_gathers(c)

            @pl.when(h + 1 < HIST)
            def _():
                pltpu.async_copy(idx_hbm.at[h + 1, pl.ds(col0, CW)],
                                 idx_v.at[c], isem[c])

            @pl.when(h >= 3)
            def _():
                wait_store(h - 3, c)

            @pl.when(h >= 1)
            def _():
                transpose(c)
                store_out(h - 1, c)
        return carry

    lax.fori_loop(0, HIST // 2, outer, 0)

    drain_gathers(1)
    wait_store(HIST - 3, 1)
    transpose(1)
    store_out(HIST - 1, 1)
    wait_store(HIST - 2, 0)
    wait_store(HIST - 1, 1)


VOCAB = 1000000
TCH = 512
NCH = VOCAB // TCH
TAIL0 = NCH * TCH
TAILC = VOCAB - TAIL0


@functools.partial(
    pl.kernel,
    mesh=_mesh,
    out_type=jax.ShapeDtypeStruct((VOCAB * EMB,), jnp.float32),
    scratch_types=[
        pltpu.VMEM((2, EMB, TCH), jnp.float32),
        pltpu.VMEM((2, TCH * EMB), jnp.float32),
        pltpu.SemaphoreType.DMA,
        pltpu.SemaphoreType.DMA,
        pltpu.SemaphoreType.DMA,
        pltpu.SemaphoreType.DMA,
    ],
    compiler_params=pltpu.CompilerParams(use_tc_tiling_on_sc=False,
                                         needs_layout_passes=False),
)
def _table_transpose(tt_hbm, out_hbm, slab, tout, isem0, isem1, osem0, osem1):
    wid = lax.axis_index("s") * NC + lax.axis_index("c")
    isem = (isem0, isem1)
    osem = (osem0, osem1)
    iota16 = jnp.arange(16, dtype=jnp.int32)

    def trans(b, nrb):
        @plsc.parallel_loop(0, nrb, unroll=2)
        def _(rb):
            rows = rb * 16 + iota16
            for dhalf in range(2):
                for d0 in range(16):
                    m = (d0 + 5 * iota16) & 15
                    dvec = dhalf * 16 + m
                    vec = plsc.load_gather(slab.at[b], [dvec, rows])
                    wpat = iota16 * EMB + dvec
                    plsc.store_scatter(tout.at[b], [wpat + rb * (16 * EMB)],
                                       vec)

    pltpu.async_copy(tt_hbm.at[:, pl.ds(wid * TCH, TCH)], slab.at[0], isem[0])
    pltpu.async_copy(tt_hbm.at[:, pl.ds((wid + 32) * TCH, TCH)], slab.at[1],
                     isem[1])

    def outer(t, carry):
        for b in range(2):
            n = t * 2 + b
            chunk = n * 32 + wid
            c0 = chunk * TCH

            @pl.when(chunk < NCH)
            def _():
                pltpu.make_async_copy(tt_hbm.at[:, pl.ds(c0, TCH)],
                                      slab.at[b], isem[b]).wait()

            @pl.when((chunk < NCH) & (n >= 2))
            def _():
                pltpu.make_async_copy(tout.at[b],
                                      out_hbm.at[pl.ds(c0 * EMB, TCH * EMB)],
                                      osem[b]).wait()

            @pl.when(chunk < NCH)
            def _():
                trans(b, TCH // 16)
                pltpu.async_copy(tout.at[b],
                                 out_hbm.at[pl.ds(c0 * EMB, TCH * EMB)],
                                 osem[b])

            @pl.when(chunk + 64 < NCH)
            def _():
                pltpu.async_copy(tt_hbm.at[:, pl.ds(c0 + 64 * TCH, TCH)],
                                 slab.at[b], isem[b])
        return carry

    lax.fori_loop(0, 32, outer, 0)

    pltpu.make_async_copy(tout.at[0], out_hbm.at[pl.ds(0, TCH * EMB)],
                          osem[0]).wait()
    pltpu.make_async_copy(tout.at[1], out_hbm.at[pl.ds(0, TCH * EMB)],
                          osem[1]).wait()

    @pl.when(wid == 0)
    def _():
        pltpu.sync_copy(tt_hbm.at[:, pl.ds(TAIL0, TAILC)],
                        slab.at[0, :, pl.ds(0, TAILC)])
        trans(0, TAILC // 16)
        pltpu.sync_copy(tout.at[0, pl.ds(0, TAILC * EMB)],
                        out_hbm.at[pl.ds(TAIL0 * EMB, TAILC * EMB)])


def kernel(inputs, table):
    idx_t = inputs.T
    table_rm = _table_transpose(table.T).reshape(VOCAB, EMB)
    r = _emb_lookup(idx_t, table_rm)
    r = r.reshape(HIST, EMB // 8, BT, 8, 128)
    return r.transpose(2, 4, 0, 1, 3).reshape(BATCH, HIST, EMB)


# speedup vs baseline: 4.9387x; 4.9387x over previous
import functools

import jax
import jax.numpy as jnp
from jax import lax
from jax.experimental import pallas as pl
from jax.experimental.pallas import tpu as pltpu
from jax.experimental.pallas import tpu_sc as plsc

EMB = 32
BATCH = 16384
HIST = 200
NC, NS = 2, 16
NW = NC * NS
BT = BATCH // 128
BT_PER_W = BT // NW
CW = BT_PER_W * 128
TW = BT_PER_W * 128 * EMB
HROW = BT * 128 * EMB

_mesh = plsc.VectorSubcoreMesh(core_axis_name="c", subcore_axis_name="s")


@functools.partial(
    pl.kernel,
    mesh=_mesh,
    out_type=jax.ShapeDtypeStruct((HIST, HROW), jnp.float32),
    scratch_types=[
        pltpu.VMEM((2, CW), jnp.int32),
        pltpu.VMEM((2, CW, EMB), jnp.float32),
        pltpu.VMEM((2, TW), jnp.float32),
        pltpu.SemaphoreType.DMA,
        pltpu.SemaphoreType.DMA,
        pltpu.SemaphoreType.DMA,
        pltpu.SemaphoreType.DMA,
        pltpu.SemaphoreType.DMA,
        pltpu.SemaphoreType.DMA,
    ],
    compiler_params=pltpu.CompilerParams(use_tc_tiling_on_sc=False,
                                         needs_layout_passes=False),
)
def _emb_lookup(idx_hbm, table_hbm, out_hbm, idx_v, g_v, t_v,
                isem0, isem1, gsem0, gsem1, osem0, osem1):
    wid = lax.axis_index("s") * NC + lax.axis_index("c")
    col0 = wid * CW
    bt0 = wid * BT_PER_W
    isem = (isem0, isem1)
    gsem = (gsem0, gsem1)
    osem = (osem0, osem1)
    iota16 = jnp.arange(16, dtype=jnp.int32)
    pat_lo = (iota16 // 8) * 4096 + (iota16 % 8) * 128
    pat_hi = pat_lo + 2 * 4096

    def fire_gathers(h, b):
        pltpu.make_async_copy(idx_hbm.at[h, pl.ds(col0, CW)], idx_v.at[b],
                              isem[b]).wait()
        for j in range(BT_PER_W):
            pltpu.async_copy(
                table_hbm.at[idx_v.at[b, pl.ds(j * 128, 128)]],
                g_v.at[b, pl.ds(j * 128, 128)], gsem[b])

    def drain_gathers(b):
        for j in range(BT_PER_W):
            pltpu.make_async_copy(
                table_hbm.at[idx_v.at[b, pl.ds(j * 128, 128)]],
                g_v.at[b, pl.ds(j * 128, 128)], gsem[b]).wait()

    def transpose(c):
        @plsc.parallel_loop(0, CW // 16, unroll=2)
        def _(rb):
            rows = rb * 16 + iota16
            s = (rb >> 3) * 1024 + (rb & 7) * 16
            for dhalf in range(2):
                for d0 in range(16):
                    m = (d0 + 5 * iota16) & 15
                    cols = dhalf * 16 + m
                    vec = plsc.load_gather(g_v.at[c], [rows, cols])
                    wpat = (dhalf * 8192 + (m // 8) * 4096
                            + (m % 8) * 128 + iota16)
                    plsc.store_scatter(t_v.at[c], [wpat + s], vec)

    def store_out(h, c):
        for dt in range(EMB // 8):
            pltpu.async_copy(
                t_v.at[c, pl.ds(dt * 4096, 4096)],
                out_hbm.at[h, pl.ds(dt * (HROW // 4) + bt0 * 1024, 4096)],
                osem[c])

    def wait_store(h, c):
        for dt in range(EMB // 8):
            pltpu.make_async_copy(
                t_v.at[c, pl.ds(dt * 4096, 4096)],
                out_hbm.at[h, pl.ds(dt * (HROW // 4) + bt0 * 1024, 4096)],
                osem[c]).wait()

    pltpu.async_copy(idx_hbm.at[0, pl.ds(col0, CW)], idx_v.at[0], isem[0])

    def outer(t, carry):
        for b in range(2):
            h = t * 2 + b
            c = 1 - b
            fire_gathers(h, b)

            @pl.when(h >= 1)
            def _():
                drain_gathers(c)

            @pl.when(h + 1 < HIST)
            def _():
                pltpu.async_copy(idx_hbm.at[h + 1, pl.ds(col0, CW)],
                                 idx_v.at[c], isem[c])

            @pl.when(h >= 3)
            def _():
                wait_store(h - 3, c)

            @pl.when(h >= 1)
            def _():
                transpose(c)
                store_out(h - 1, c)
        return carry

    lax.fori_loop(0, HIST // 2, outer, 0)

    drain_gathers(1)
    wait_store(HIST - 3, 1)
    transpose(1)
    store_out(HIST - 1, 1)
    wait_store(HIST - 2, 0)
    wait_store(HIST - 1, 1)


VOCAB = 1000000
VPAD = 1000064
NTC = VPAD // 128
TCH = 512
NCH = (NTC - 1) // 4
TFLAT = 4 * NTC * 8 * 128


@functools.partial(
    pl.kernel,
    mesh=_mesh,
    out_type=jax.ShapeDtypeStruct((VOCAB * EMB,), jnp.float32),
    scratch_types=[
        pltpu.VMEM((2, 4 * 4096), jnp.float32),
        pltpu.VMEM((2, TCH * EMB), jnp.float32),
        pltpu.SemaphoreType.DMA,
        pltpu.SemaphoreType.DMA,
        pltpu.SemaphoreType.DMA,
        pltpu.SemaphoreType.DMA,
    ],
    compiler_params=pltpu.CompilerParams(use_tc_tiling_on_sc=False,
                                         needs_layout_passes=False),
)
def _table_transpose(t6_hbm, out_hbm, slab, tout, isem0, isem1, osem0, osem1):
    wid = lax.axis_index("s") * NC + lax.axis_index("c")
    isem = (isem0, isem1)
    osem = (osem0, osem1)
    iota16 = jnp.arange(16, dtype=jnp.int32)

    def fetch(b, tc0):
        for tr in range(4):
            pltpu.async_copy(
                t6_hbm.at[pl.ds((tr * NTC) * 1024 + tc0 * 1024, 4096)],
                slab.at[b, pl.ds(tr * 4096, 4096)], isem[b])

    def fetch_wait(b, tc0):
        for tr in range(4):
            pltpu.make_async_copy(
                t6_hbm.at[pl.ds((tr * NTC) * 1024 + tc0 * 1024, 4096)],
                slab.at[b, pl.ds(tr * 4096, 4096)], isem[b]).wait()

    def trans(b, nrb):
        @plsc.parallel_loop(0, nrb, unroll=2)
        def _(rb):
            s_rd = (rb >> 3) * 1024 + (rb & 7) * 16
            s_wr = rb * (16 * EMB)
            for dhalf in range(2):
                for d0 in range(16):
                    m = (d0 + 5 * iota16) & 15
                    rpat = ((dhalf * 2 + (m >> 3)) * 4096 + (m & 7) * 128
                            + iota16)
                    vec = plsc.load_gather(slab.at[b], [rpat + s_rd])
                    wpat = iota16 * EMB + dhalf * 16 + m
                    plsc.store_scatter(tout.at[b], [wpat + s_wr], vec)

    fetch(0, wid * 4)
    fetch(1, (wid + 32) * 4)

    def outer(t, carry):
        for b in range(2):
            n = t * 2 + b
            chunk = n * 32 + wid

            @pl.when(chunk < NCH)
            def _():
                fetch_wait(b, chunk * 4)

            @pl.when((chunk < NCH) & (n >= 2))
            def _():
                pltpu.make_async_copy(
                    tout.at[b],
                    out_hbm.at[pl.ds(chunk * (TCH * EMB), TCH * EMB)],
                    osem[b]).wait()

            @pl.when(chunk < NCH)
            def _():
                trans(b, TCH // 16)
                pltpu.async_copy(
                    tout.at[b],
                    out_hbm.at[pl.ds(chunk * (TCH * EMB), TCH * EMB)],
                    osem[b])

            @pl.when(chunk + 64 < NCH)
            def _():
                fetch(b, (chunk + 64) * 4)
        return carry

    lax.fori_loop(0, 32, outer, 0)

    pltpu.make_async_copy(tout.at[0], out_hbm.at[pl.ds(0, TCH * EMB)],
                          osem[0]).wait()
    pltpu.make_async_copy(tout.at[1], out_hbm.at[pl.ds(0, TCH * EMB)],
                          osem[1]).wait()

    @pl.when(wid == 0)
    def _():
        for tr in range(4):
            pltpu.sync_copy(
                t6_hbm.at[pl.ds((tr * NTC + NCH * 4) * 1024, 1024)],
                slab.at[0, pl.ds(tr * 4096, 1024)])
        trans(0, 4)
        pltpu.sync_copy(tout.at[0, pl.ds(0, 64 * EMB)],
                        out_hbm.at[pl.ds(NCH * (TCH * EMB), 64 * EMB)])


def kernel(inputs, table):
    idx_t = inputs.T
    tp = jnp.pad(table, ((0, VPAD - VOCAB), (0, 0)))
    t6 = (tp.T.reshape(4, 8, NTC, 128).transpose(0, 2, 1, 3)
          .reshape(TFLAT))
    table_rm = _table_transpose(t6).reshape(VOCAB, EMB)
    r = _emb_lookup(idx_t, table_rm)
    r = r.reshape(HIST, EMB // 8, BT, 8, 128)
    return r.transpose(2, 4, 0, 1, 3).reshape(BATCH, HIST, EMB)


# speedup vs baseline: 4.9639x; 1.0051x over previous
import functools

import jax
import jax.numpy as jnp
from jax import lax
from jax.experimental import pallas as pl
from jax.experimental.pallas import tpu as pltpu
from jax.experimental.pallas import tpu_sc as plsc

EMB = 32
BATCH = 16384
HIST = 200
NC, NS = 2, 16
NW = NC * NS
BT = BATCH // 128
BT_PER_W = BT // NW
CW = BT_PER_W * 128
TW = BT_PER_W * 128 * EMB
HROW = BT * 128 * EMB

_mesh = plsc.VectorSubcoreMesh(core_axis_name="c", subcore_axis_name="s")


@functools.partial(
    pl.kernel,
    mesh=_mesh,
    out_type=jax.ShapeDtypeStruct((HIST, HROW), jnp.float32),
    scratch_types=[
        pltpu.VMEM((2, CW), jnp.int32),
        pltpu.VMEM((2, CW, EMB), jnp.float32),
        pltpu.VMEM((2, TW), jnp.float32),
        pltpu.SemaphoreType.DMA,
        pltpu.SemaphoreType.DMA,
        pltpu.SemaphoreType.DMA,
        pltpu.SemaphoreType.DMA,
        pltpu.SemaphoreType.DMA,
        pltpu.SemaphoreType.DMA,
    ],
    compiler_params=pltpu.CompilerParams(use_tc_tiling_on_sc=False,
                                         needs_layout_passes=False),
)
def _emb_lookup(idx_hbm, table_hbm, out_hbm, idx_v, g_v, t_v,
                isem0, isem1, gsem0, gsem1, osem0, osem1):
    wid = lax.axis_index("s") * NC + lax.axis_index("c")
    col0 = wid * CW
    bt0 = wid * BT_PER_W
    isem = (isem0, isem1)
    gsem = (gsem0, gsem1)
    osem = (osem0, osem1)
    iota16 = jnp.arange(16, dtype=jnp.int32)
    pat_lo = (iota16 // 8) * 4096 + (iota16 % 8) * 128
    pat_hi = pat_lo + 2 * 4096

    def fire_gathers(h, b):
        pltpu.make_async_copy(idx_hbm.at[h, pl.ds(col0, CW)], idx_v.at[b],
                              isem[b]).wait()
        for j in range(BT_PER_W):
            pltpu.async_copy(
                table_hbm.at[idx_v.at[b, pl.ds(j * 128, 128)]],
                g_v.at[b, pl.ds(j * 128, 128)], gsem[b])

    def drain_gathers(b):
        for j in range(BT_PER_W):
            pltpu.make_async_copy(
                table_hbm.at[idx_v.at[b, pl.ds(j * 128, 128)]],
                g_v.at[b, pl.ds(j * 128, 128)], gsem[b]).wait()

    def transpose(c):
        @plsc.parallel_loop(0, CW // 16, unroll=2)
        def _(rb):
            rows = rb * 16 + iota16
            s = (rb >> 3) * 1024 + (rb & 7) * 16
            for dhalf in range(2):
                for d0 in range(16):
                    m = (d0 + 5 * iota16) & 15
                    cols = dhalf * 16 + m
                    vec = plsc.load_gather(g_v.at[c], [rows, cols])
                    wpat = (dhalf * 8192 + (m // 8) * 4096
                            + (m % 8) * 128 + iota16)
                    plsc.store_scatter(t_v.at[c], [wpat + s], vec)

    def store_out(h, c):
        for dt in range(EMB // 8):
            pltpu.async_copy(
                t_v.at[c, pl.ds(dt * 4096, 4096)],
                out_hbm.at[h, pl.ds(dt * (HROW // 4) + bt0 * 1024, 4096)],
                osem[c])

    def wait_store(h, c):
        for dt in range(EMB // 8):
            pltpu.make_async_copy(
                t_v.at[c, pl.ds(dt * 4096, 4096)],
                out_hbm.at[h, pl.ds(dt * (HROW // 4) + bt0 * 1024, 4096)],
                osem[c]).wait()

    pltpu.async_copy(idx_hbm.at[0, pl.ds(col0, CW)], idx_v.at[0], isem[0])

    def outer(t, carry):
        for b in range(2):
            h = t * 2 + b
            c = 1 - b
            fire_gathers(h, b)

            @pl.when(h >= 1)
            def _():
                drain_gathers(c)

            @pl.when(h + 1 < HIST)
            def _():
                pltpu.async_copy(idx_hbm.at[h + 1, pl.ds(col0, CW)],
                                 idx_v.at[c], isem[c])

            @pl.when(h >= 3)
            def _():
                wait_store(h - 3, c)

            @pl.when(h >= 1)
            def _():
                transpose(c)
                store_out(h - 1, c)
        return carry

    lax.fori_loop(0, HIST // 2, outer, 0)

    drain_gathers(1)
    wait_store(HIST - 3, 1)
    transpose(1)
    store_out(HIST - 1, 1)
    wait_store(HIST - 2, 0)
    wait_store(HIST - 1, 1)


VOCAB = 1000000
VPAD = 1000064
NTC = VPAD // 128
TCH = 512
NTC2 = NTC - 1
NCH = NTC2 // 4
TFLAT = 4 * NTC2 * 8 * 128


@functools.partial(
    pl.kernel,
    mesh=_mesh,
    out_type=jax.ShapeDtypeStruct((VOCAB * EMB,), jnp.float32),
    scratch_types=[
        pltpu.VMEM((2, 4 * 4096), jnp.float32),
        pltpu.VMEM((2, TCH * EMB), jnp.float32),
        pltpu.SemaphoreType.DMA,
        pltpu.SemaphoreType.DMA,
        pltpu.SemaphoreType.DMA,
        pltpu.SemaphoreType.DMA,
    ],
    compiler_params=pltpu.CompilerParams(use_tc_tiling_on_sc=False,
                                         needs_layout_passes=False),
)
def _table_transpose(t6_hbm, tail_hbm, out_hbm, slab, tout,
                     isem0, isem1, osem0, osem1):
    wid = lax.axis_index("s") * NC + lax.axis_index("c")
    isem = (isem0, isem1)
    osem = (osem0, osem1)
    iota16 = jnp.arange(16, dtype=jnp.int32)

    def fetch(b, tc0):
        for tr in range(4):
            pltpu.async_copy(
                t6_hbm.at[pl.ds((tr * NTC2) * 1024 + tc0 * 1024, 4096)],
                slab.at[b, pl.ds(tr * 4096, 4096)], isem[b])

    def fetch_wait(b, tc0):
        for tr in range(4):
            pltpu.make_async_copy(
                t6_hbm.at[pl.ds((tr * NTC2) * 1024 + tc0 * 1024, 4096)],
                slab.at[b, pl.ds(tr * 4096, 4096)], isem[b]).wait()

    def trans(b, nrb):
        @plsc.parallel_loop(0, nrb, unroll=2)
        def _(rb):
            s_rd = (rb >> 3) * 1024 + (rb & 7) * 16
            s_wr = rb * (16 * EMB)
            for dhalf in range(2):
                for d0 in range(16):
                    m = (d0 + 5 * iota16) & 15
                    rpat = ((dhalf * 2 + (m >> 3)) * 4096 + (m & 7) * 128
                            + iota16)
                    vec = plsc.load_gather(slab.at[b], [rpat + s_rd])
                    wpat = iota16 * EMB + dhalf * 16 + m
                    plsc.store_scatter(tout.at[b], [wpat + s_wr], vec)

    fetch(0, wid * 4)
    fetch(1, (wid + 32) * 4)

    def outer(t, carry):
        for b in range(2):
            n = t * 2 + b
            chunk = n * 32 + wid

            @pl.when(chunk < NCH)
            def _():
                fetch_wait(b, chunk * 4)

            @pl.when((chunk < NCH) & (n >= 2))
            def _():
                pltpu.make_async_copy(
                    tout.at[b],
                    out_hbm.at[pl.ds(chunk * (TCH * EMB), TCH * EMB)],
                    osem[b]).wait()

            @pl.when(chunk < NCH)
            def _():
                trans(b, TCH // 16)
                pltpu.async_copy(
                    tout.at[b],
                    out_hbm.at[pl.ds(chunk * (TCH * EMB), TCH * EMB)],
                    osem[b])

            @pl.when(chunk + 64 < NCH)
            def _():
                fetch(b, (chunk + 64) * 4)
        return carry

    lax.fori_loop(0, 32, outer, 0)

    pltpu.make_async_copy(tout.at[0], out_hbm.at[pl.ds(0, TCH * EMB)],
                          osem[0]).wait()
    pltpu.make_async_copy(tout.at[1], out_hbm.at[pl.ds(0, TCH * EMB)],
                          osem[1]).wait()

    @pl.when(wid == 0)
    def _():
        pltpu.sync_copy(tail_hbm, tout.at[0, pl.ds(0, 64 * EMB)])
        pltpu.sync_copy(tout.at[0, pl.ds(0, 64 * EMB)],
                        out_hbm.at[pl.ds(NCH * (TCH * EMB), 64 * EMB)])


def kernel(inputs, table):
    idx_t = inputs.T
    tp = table[:NCH * TCH]
    t6 = (tp.T.reshape(4, 8, NTC2, 128).transpose(0, 2, 1, 3)
          .reshape(TFLAT))
    tail64 = table[NCH * TCH:].reshape(64 * EMB)
    table_rm = _table_transpose(t6, tail64).reshape(VOCAB, EMB)
    r = _emb_lookup(idx_t, table_rm)
    r = r.reshape(HIST, EMB // 8, BT, 8, 128)
    return r.transpose(2, 4, 0, 1, 3).reshape(BATCH, HIST, EMB)
